# baseline (speedup 1.0000x reference)
import functools
import jax
import jax.numpy as jnp
from jax import lax
from jax.experimental import pallas as pl
from jax.experimental.pallas import tpu as pltpu, tpu_sc as plsc

_MAX_LEN = 8192
_DIM = 768
_BATCH = 2
_NW = 32
_ROWS_PER_W = _MAX_LEN // _NW
_CHUNK_ROWS = 64
_CH = _CHUNK_ROWS * _DIM
_N_CHUNKS = _ROWS_PER_W // _CHUNK_ROWS


def _sc_kernel(x_hbm, pos_hbm, out_hbm, x_v, pos_v):
    wid = lax.axis_index("s") * 2 + lax.axis_index("c")
    base = wid * _ROWS_PER_W * _DIM

    def body(i16):
        x_v[pl.ds(i16 * 16, 16)] = x_v[pl.ds(i16 * 16, 16)] + pos_v[pl.ds(i16 * 16, 16)]

    for ci in range(_N_CHUNKS):
        off = base + ci * _CH
        pltpu.sync_copy(pos_hbm.at[pl.ds(off, _CH)], pos_v)
        for b in range(_BATCH):
            pltpu.sync_copy(x_hbm.at[b, pl.ds(off, _CH)], x_v)
            lax.fori_loop(0, _CH // 16, lambda i, _: (body(i), None)[1], None)
            pltpu.sync_copy(x_v, out_hbm.at[b, pl.ds(off, _CH)])


def kernel(x, pos_table):
    batch, max_len, dim = x.shape
    x_flat = x.reshape(batch, max_len * dim)
    pos_flat = pos_table.reshape(max_len * dim)
    mesh = plsc.VectorSubcoreMesh(core_axis_name="c", subcore_axis_name="s")
    out = pl.kernel(
        _sc_kernel,
        mesh=mesh,
        out_type=jax.ShapeDtypeStruct((batch, max_len * dim), jnp.float32),
        scratch_types=[
            pltpu.VMEM((_CH,), jnp.float32),
            pltpu.VMEM((_CH,), jnp.float32),
        ],
    )(x_flat, pos_flat)
    return out.reshape(batch, max_len, dim)


# speedup vs baseline: 1.0175x; 1.0175x over previous
import jax
import jax.numpy as jnp
from jax import lax
from jax.experimental import pallas as pl
from jax.experimental.pallas import tpu as pltpu, tpu_sc as plsc

_MAX_LEN = 8192
_DIM = 768
_BATCH = 2
_NW = 32
_ROWS_PER_W = _MAX_LEN // _NW
_CHUNK_ROWS = 16
_CH = _CHUNK_ROWS * _DIM
_N_CHUNKS = _ROWS_PER_W // _CHUNK_ROWS
_N_ITEMS = _N_CHUNKS * _BATCH
_UNROLL = 8


def _sc_kernel(x_hbm, pos_hbm, out_hbm, x_vs, pos_vs,
               in_sems, out_sems, pos_sems):
    wid = lax.axis_index("s") * 2 + lax.axis_index("c")
    base = wid * _ROWS_PER_W * _DIM

    def x_in(k):
        ci, b, xb = k // 2, k % 2, k % 4
        pltpu.async_copy(x_hbm.at[b, pl.ds(base + ci * _CH, _CH)],
                         x_vs.at[xb], in_sems.at[xb])

    def pos_in(ci):
        pltpu.async_copy(pos_hbm.at[pl.ds(base + ci * _CH, _CH)],
                         pos_vs.at[ci % 2], pos_sems.at[ci % 2])

    pos_in(0)
    pos_in(1)
    x_in(0)
    x_in(1)

    for k in range(_N_ITEMS):
        ci, b, xb, pb = k // 2, k % 2, k % 4, (k // 2) % 2
        pltpu.make_async_copy(x_hbm.at[b, pl.ds(base + ci * _CH, _CH)],
                              x_vs.at[xb], in_sems.at[xb]).wait()
        if b == 0:
            pltpu.make_async_copy(pos_hbm.at[pl.ds(base + ci * _CH, _CH)],
                                  pos_vs.at[pb], pos_sems.at[pb]).wait()

        def body(i, _, xb=xb, pb=pb):
            for u in range(_UNROLL):
                o = i * (_UNROLL * 16) + u * 16
                x_vs[xb, pl.ds(o, 16)] = (x_vs[xb, pl.ds(o, 16)]
                                          + pos_vs[pb, pl.ds(o, 16)])
            return 0

        lax.fori_loop(0, _CH // (16 * _UNROLL), body, 0)

        pltpu.async_copy(x_vs.at[xb],
                         out_hbm.at[b, pl.ds(base + ci * _CH, _CH)],
                         out_sems.at[xb])

        if b == 1 and ci + 2 < _N_CHUNKS:
            pos_in(ci + 2)

        kn = k + 2
        if kn < _N_ITEMS:
            if kn >= 4:
                cp, bp = (kn - 4) // 2, (kn - 4) % 2
                pltpu.make_async_copy(
                    x_vs.at[kn % 4],
                    out_hbm.at[bp, pl.ds(base + cp * _CH, _CH)],
                    out_sems.at[kn % 4]).wait()
            x_in(kn)

    for k in range(_N_ITEMS - 4, _N_ITEMS):
        ci, b = k // 2, k % 2
        pltpu.make_async_copy(x_vs.at[k % 4],
                              out_hbm.at[b, pl.ds(base + ci * _CH, _CH)],
                              out_sems.at[k % 4]).wait()


def kernel(x, pos_table):
    batch, max_len, dim = x.shape
    x_flat = x.reshape(batch, max_len * dim)
    pos_flat = pos_table.reshape(max_len * dim)
    mesh = plsc.VectorSubcoreMesh(core_axis_name="c", subcore_axis_name="s")
    out = pl.kernel(
        _sc_kernel,
        mesh=mesh,
        out_type=jax.ShapeDtypeStruct((batch, max_len * dim), jnp.float32),
        scratch_types=[
            pltpu.VMEM((4, _CH), jnp.float32),
            pltpu.VMEM((2, _CH), jnp.float32),
            pltpu.SemaphoreType.DMA((4,)),
            pltpu.SemaphoreType.DMA((4,)),
            pltpu.SemaphoreType.DMA((2,)),
        ],
    )(x_flat, pos_flat)
    return out.reshape(batch, max_len, dim)


# speedup vs baseline: 6.8224x; 6.7047x over previous
import jax
import jax.numpy as jnp
from jax.experimental import pallas as pl

_CHUNK = 1024


def _add_kernel(x_ref, pos_ref, out_ref):
    out_ref[...] = x_ref[...] + pos_ref[...][None, :, :]


def kernel(x, pos_table):
    batch, max_len, dim = x.shape
    grid = (max_len // _CHUNK,)
    return pl.pallas_call(
        _add_kernel,
        grid=grid,
        in_specs=[
            pl.BlockSpec((batch, _CHUNK, dim), lambda i: (0, i, 0)),
            pl.BlockSpec((_CHUNK, dim), lambda i: (i, 0)),
        ],
        out_specs=pl.BlockSpec((batch, _CHUNK, dim), lambda i: (0, i, 0)),
        out_shape=jax.ShapeDtypeStruct(x.shape, x.dtype),
    )(x, pos_table)


# speedup vs baseline: 6.8253x; 1.0004x over previous
import jax
import jax.numpy as jnp
from jax.experimental import pallas as pl

_CHUNK = 2048
_DSPLIT = 2


def _add_kernel(x_ref, pos_ref, out_ref):
    out_ref[...] = x_ref[...] + pos_ref[...][None, :, :]


def kernel(x, pos_table):
    batch, max_len, dim = x.shape
    dblk = dim // _DSPLIT
    grid = (max_len // _CHUNK, _DSPLIT)
    return pl.pallas_call(
        _add_kernel,
        grid=grid,
        in_specs=[
            pl.BlockSpec((batch, _CHUNK, dblk), lambda i, j: (0, i, j)),
            pl.BlockSpec((_CHUNK, dblk), lambda i, j: (i, j)),
        ],
        out_specs=pl.BlockSpec((batch, _CHUNK, dblk), lambda i, j: (0, i, j)),
        out_shape=jax.ShapeDtypeStruct(x.shape, x.dtype),
    )(x, pos_table)
